# Initial kernel scaffold; baseline (speedup 1.0000x reference)
import jax
import jax.numpy as jnp
from jax.experimental import pallas as pl


def kernel(sentence_tensor, tag_string_tensor, word_table, tag_table):
    raise NotImplementedError("write your pallas kernel here")


# speedup vs baseline: 91.3700x; 91.3700x over previous
import jax
import jax.numpy as jnp
from jax import lax
from jax.experimental import pallas as pl
from jax.experimental.pallas import tpu as pltpu, tpu_sc as plsc

_info = plsc.get_sparse_core_info()
_NC, _NS, _L = _info.num_cores, _info.num_subcores, _info.num_lanes
_NW = _NC * _NS

_N = 16384 * 200
_PER_W = _N // _NW
_CHUNK = 12800
_NCHUNK = _PER_W // _CHUNK


def _body(sent_hbm, tagids_hbm, wtbl_hbm, ttbl_hbm, out_tok_hbm, out_tag_hbm,
          widx_v, wout_v, tidx_v, tout_v, ttbl_v, sem):
    wid = lax.axis_index("s") * _NC + lax.axis_index("c")
    base = wid * _PER_W

    pltpu.sync_copy(ttbl_hbm, ttbl_v)

    def chunk_body(j, carry):
        off = pl.multiple_of(base + j * _CHUNK, _CHUNK)
        pltpu.sync_copy(sent_hbm.at[pl.ds(off, _CHUNK)], widx_v)
        pltpu.async_copy(wtbl_hbm.at[widx_v], wout_v, sem).wait()
        pltpu.sync_copy(wout_v, out_tok_hbm.at[pl.ds(off, _CHUNK)])
        pltpu.sync_copy(tagids_hbm.at[pl.ds(off, _CHUNK)], tidx_v)

        def vec_body(i, c):
            idx = tidx_v[pl.ds(i * _L, _L)]
            tout_v[pl.ds(i * _L, _L)] = plsc.load_gather(ttbl_v, [idx])
            return c

        lax.fori_loop(0, _CHUNK // _L, vec_body, 0)
        pltpu.sync_copy(tout_v, out_tag_hbm.at[pl.ds(off, _CHUNK)])
        return carry

    lax.fori_loop(0, _NCHUNK, chunk_body, 0)


def kernel(sentence_tensor, tag_string_tensor, word_table, tag_table):
    shape = sentence_tensor.shape
    sent = sentence_tensor.reshape(_N)
    tags = tag_string_tensor.reshape(_N)

    mesh = plsc.VectorSubcoreMesh(core_axis_name="c", subcore_axis_name="s")
    k = pl.kernel(
        _body,
        mesh=mesh,
        compiler_params=pltpu.CompilerParams(needs_layout_passes=False),
        out_type=[
            jax.ShapeDtypeStruct((_N,), jnp.int32),
            jax.ShapeDtypeStruct((_N,), jnp.int32),
        ],
        scratch_types=[
            pltpu.VMEM((_CHUNK,), jnp.int32),
            pltpu.VMEM((_CHUNK,), jnp.int32),
            pltpu.VMEM((_CHUNK,), jnp.int32),
            pltpu.VMEM((_CHUNK,), jnp.int32),
            pltpu.VMEM((64,), jnp.int32),
            pltpu.SemaphoreType.DMA,
        ],
    )
    tok, tag = k(sent, tags, word_table, tag_table)
    return (tok.reshape(shape), tag.reshape(shape))


# speedup vs baseline: 107.6145x; 1.1778x over previous
import jax
import jax.numpy as jnp
from jax import lax
from jax.experimental import pallas as pl
from jax.experimental.pallas import tpu as pltpu, tpu_sc as plsc

_info = plsc.get_sparse_core_info()
_NC, _NS, _L = _info.num_cores, _info.num_subcores, _info.num_lanes
_NW = _NC * _NS

_N = 16384 * 200
_PER_W = _N // _NW
_CHUNK = 12800
_NCHUNK = _PER_W // _CHUNK


def _body(sent_hbm, tagids_hbm, wtbl_hbm, ttbl_hbm, out_tok_hbm, out_tag_hbm,
          widx0, widx1, wout0, wout1, tidx0, tidx1, tout0, tout1, ttbl_v,
          sem_wi0, sem_wi1, sem_ti0, sem_ti1, sem_g0, sem_g1,
          sem_sw0, sem_sw1, sem_st0, sem_st1):
    wid = lax.axis_index("s") * _NC + lax.axis_index("c")
    base = wid * _PER_W

    widx = [widx0, widx1]
    wout = [wout0, wout1]
    tidx = [tidx0, tidx1]
    tout = [tout0, tout1]
    sem_wi = [sem_wi0, sem_wi1]
    sem_ti = [sem_ti0, sem_ti1]
    sem_g = [sem_g0, sem_g1]
    sem_sw = [sem_sw0, sem_sw1]
    sem_st = [sem_st0, sem_st1]

    pltpu.sync_copy(ttbl_hbm, ttbl_v)

    offs = [pl.multiple_of(base + j * _CHUNK, _CHUNK) for j in range(_NCHUNK)]

    d_ti = [None, None]
    d_wi = [None, None]
    d_g = [None, None]
    d_sw = [None, None]
    d_st = [None, None]

    d_wi[0] = pltpu.async_copy(sent_hbm.at[pl.ds(offs[0], _CHUNK)], widx[0],
                               sem_wi[0])
    d_ti[0] = pltpu.async_copy(tagids_hbm.at[pl.ds(offs[0], _CHUNK)], tidx[0],
                               sem_ti[0])
    d_wi[0].wait()
    d_g[0] = pltpu.async_copy(wtbl_hbm.at[widx[0]], wout[0], sem_g[0])

    for j in range(_NCHUNK):
        b = j & 1
        nb = b ^ 1
        if j + 1 < _NCHUNK:
            d_wi[nb] = pltpu.async_copy(
                sent_hbm.at[pl.ds(offs[j + 1], _CHUNK)], widx[nb], sem_wi[nb])
            d_ti[nb] = pltpu.async_copy(
                tagids_hbm.at[pl.ds(offs[j + 1], _CHUNK)], tidx[nb],
                sem_ti[nb])

        d_ti[b].wait()
        if d_st[b] is not None:
            d_st[b].wait()
            d_st[b] = None

        tidx_b, tout_b = tidx[b], tout[b]

        def vec_body(i, c):
            idx = tidx_b[pl.ds(i * _L, _L)]
            tout_b[pl.ds(i * _L, _L)] = plsc.load_gather(ttbl_v, [idx])
            return c

        lax.fori_loop(0, _CHUNK // _L, vec_body, 0)
        d_st[b] = pltpu.async_copy(tout[b], out_tag_hbm.at[pl.ds(offs[j], _CHUNK)],
                                   sem_st[b])

        d_g[b].wait()
        d_sw[b] = pltpu.async_copy(wout[b], out_tok_hbm.at[pl.ds(offs[j], _CHUNK)],
                                   sem_sw[b])

        if j + 1 < _NCHUNK:
            d_wi[nb].wait()
            if d_sw[nb] is not None:
                d_sw[nb].wait()
                d_sw[nb] = None
            d_g[nb] = pltpu.async_copy(wtbl_hbm.at[widx[nb]], wout[nb],
                                       sem_g[nb])

    for d in (*d_sw, *d_st):
        if d is not None:
            d.wait()


def kernel(sentence_tensor, tag_string_tensor, word_table, tag_table):
    shape = sentence_tensor.shape
    sent = sentence_tensor.reshape(_N)
    tags = tag_string_tensor.reshape(_N)

    mesh = plsc.VectorSubcoreMesh(core_axis_name="c", subcore_axis_name="s")
    k = pl.kernel(
        _body,
        mesh=mesh,
        compiler_params=pltpu.CompilerParams(needs_layout_passes=False),
        out_type=[
            jax.ShapeDtypeStruct((_N,), jnp.int32),
            jax.ShapeDtypeStruct((_N,), jnp.int32),
        ],
        scratch_types=[
            pltpu.VMEM((_CHUNK,), jnp.int32),
            pltpu.VMEM((_CHUNK,), jnp.int32),
            pltpu.VMEM((_CHUNK,), jnp.int32),
            pltpu.VMEM((_CHUNK,), jnp.int32),
            pltpu.VMEM((_CHUNK,), jnp.int32),
            pltpu.VMEM((_CHUNK,), jnp.int32),
            pltpu.VMEM((_CHUNK,), jnp.int32),
            pltpu.VMEM((_CHUNK,), jnp.int32),
            pltpu.VMEM((64,), jnp.int32),
        ] + [pltpu.SemaphoreType.DMA] * 10,
    )
    tok, tag = k(sent, tags, word_table, tag_table)
    return (tok.reshape(shape), tag.reshape(shape))


# speedup vs baseline: 121.2203x; 1.1264x over previous
import jax
import jax.numpy as jnp
from jax import lax
from jax.experimental import pallas as pl
from jax.experimental.pallas import tpu as pltpu, tpu_sc as plsc

_info = plsc.get_sparse_core_info()
_NC, _NS, _L = _info.num_cores, _info.num_subcores, _info.num_lanes
_NW = _NC * _NS

_ROWS = 16384
_COLS = 200
_ROWS_W = _ROWS // _NW
_G = 32
_NCHUNK = _ROWS_W // _G
_FLAT = _G * _COLS

_WINS = [16 * w for w in range(12)] + [_COLS - _L]


def _body(sent_hbm, tagids_hbm, wtbl_hbm, ttbl_hbm, out_tok_hbm, out_tag_hbm,
          widx0, widx1, wout0, wout1, tidx0, tidx1, tout0, tout1,
          fidx0, fidx1, fout0, fout1, ttbl_v,
          sem_wi0, sem_wi1, sem_ti0, sem_ti1, sem_g,
          sem_sw0, sem_sw1, sem_st0, sem_st1):
    wid = lax.axis_index("s") * _NC + lax.axis_index("c")
    row0 = wid * _ROWS_W

    widx = [widx0, widx1]
    wout = [wout0, wout1]
    tidx = [tidx0, tidx1]
    tout = [tout0, tout1]
    fidx = [fidx0, fidx1]
    fout = [fout0, fout1]
    sem_wi = [sem_wi0, sem_wi1]
    sem_ti = [sem_ti0, sem_ti1]
    sem_sw = [sem_sw0, sem_sw1]
    sem_st = [sem_st0, sem_st1]

    pltpu.sync_copy(ttbl_hbm, ttbl_v)

    r0s = [row0 + j * _G for j in range(_NCHUNK)]

    d_ti = [None, None]
    d_wi = [None, None]
    d_g = [None, None]
    d_sw = [None, None]
    d_st = [None, None]

    d_wi[0] = pltpu.async_copy(sent_hbm.at[pl.ds(r0s[0], _G), :], widx[0],
                               sem_wi[0])
    d_ti[0] = pltpu.async_copy(tagids_hbm.at[pl.ds(r0s[0], _G), :], tidx[0],
                               sem_ti[0])

    for j in range(_NCHUNK):
        b = j & 1
        nb = b ^ 1
        if j + 1 < _NCHUNK:
            d_wi[nb] = pltpu.async_copy(
                sent_hbm.at[pl.ds(r0s[j + 1], _G), :], widx[nb], sem_wi[nb])
            d_ti[nb] = pltpu.async_copy(
                tagids_hbm.at[pl.ds(r0s[j + 1], _G), :], tidx[nb], sem_ti[nb])

        d_wi[b].wait()
        widx_b, fidx_b = widx[b], fidx[b]

        def pack(r, c):
            for cs in _WINS:
                fidx_b[pl.ds(r * _COLS + cs, _L)] = widx_b[r, pl.ds(cs, _L)]
            return c

        lax.fori_loop(0, _G, pack, 0)
        d_g[b] = pltpu.async_copy(wtbl_hbm.at[fidx_b], fout[b], sem_g)

        d_ti[b].wait()
        if d_st[b] is not None:
            d_st[b].wait()
            d_st[b] = None
        tidx_b, tout_b = tidx[b], tout[b]

        def tag_rows(r, c):
            for cs in _WINS:
                idx = tidx_b[r, pl.ds(cs, _L)]
                tout_b[r, pl.ds(cs, _L)] = plsc.load_gather(ttbl_v, [idx])
            return c

        lax.fori_loop(0, _G, tag_rows, 0)
        d_st[b] = pltpu.async_copy(tout[b],
                                   out_tag_hbm.at[pl.ds(r0s[j], _G), :],
                                   sem_st[b])

        d_g[b].wait()
        d_g[b] = None
        if d_sw[b] is not None:
            d_sw[b].wait()
            d_sw[b] = None
        wout_b, fout_b = wout[b], fout[b]

        def unpack(r, c):
            for cs in _WINS:
                wout_b[r, pl.ds(cs, _L)] = fout_b[pl.ds(r * _COLS + cs, _L)]
            return c

        lax.fori_loop(0, _G, unpack, 0)
        d_sw[b] = pltpu.async_copy(wout[b],
                                   out_tok_hbm.at[pl.ds(r0s[j], _G), :],
                                   sem_sw[b])

    for d in (*d_sw, *d_st):
        if d is not None:
            d.wait()


def kernel(sentence_tensor, tag_string_tensor, word_table, tag_table):
    shape = sentence_tensor.shape

    mesh = plsc.VectorSubcoreMesh(core_axis_name="c", subcore_axis_name="s")
    k = pl.kernel(
        _body,
        mesh=mesh,
        compiler_params=pltpu.CompilerParams(needs_layout_passes=False),
        out_type=[
            jax.ShapeDtypeStruct(shape, jnp.int32),
            jax.ShapeDtypeStruct(shape, jnp.int32),
        ],
        scratch_types=[
            pltpu.VMEM((_G, _COLS), jnp.int32),
            pltpu.VMEM((_G, _COLS), jnp.int32),
            pltpu.VMEM((_G, _COLS), jnp.int32),
            pltpu.VMEM((_G, _COLS), jnp.int32),
            pltpu.VMEM((_G, _COLS), jnp.int32),
            pltpu.VMEM((_G, _COLS), jnp.int32),
            pltpu.VMEM((_G, _COLS), jnp.int32),
            pltpu.VMEM((_G, _COLS), jnp.int32),
            pltpu.VMEM((_FLAT,), jnp.int32),
            pltpu.VMEM((_FLAT,), jnp.int32),
            pltpu.VMEM((_FLAT,), jnp.int32),
            pltpu.VMEM((_FLAT,), jnp.int32),
            pltpu.VMEM((64,), jnp.int32),
        ] + [pltpu.SemaphoreType.DMA] * 9,
    )
    tok, tag = k(sentence_tensor, tag_string_tensor, word_table, tag_table)
    return (tok, tag)


# speedup vs baseline: 128.0630x; 1.0564x over previous
import jax
import jax.numpy as jnp
from jax import lax
from jax.experimental import pallas as pl
from jax.experimental.pallas import tpu as pltpu, tpu_sc as plsc

_info = plsc.get_sparse_core_info()
_NC, _NS, _L = _info.num_cores, _info.num_subcores, _info.num_lanes
_NW = _NC * _NS

_ROWS = 16384
_COLS = 200
_ROWS_W = _ROWS // _NW
_G = 32
_NCHUNK = _ROWS_W // _G
_FLAT = _G * _COLS

_WINS = [16 * w for w in range(12)] + [_COLS - _L]


def _body(sent_hbm, tagids_hbm, wtbl_hbm, ttbl_hbm, out_tok_hbm, out_tag_hbm,
          widx0, widx1, wout0, wout1, tidx0, tidx1, tout0, tout1,
          fidx0, fidx1, fout0, fout1, ttbl_v,
          sem_wi0, sem_wi1, sem_ti0, sem_ti1, sem_g,
          sem_sw0, sem_sw1, sem_st0, sem_st1):
    wid = lax.axis_index("s") * _NC + lax.axis_index("c")
    row0 = wid * _ROWS_W

    widx = [widx0, widx1]
    wout = [wout0, wout1]
    tidx = [tidx0, tidx1]
    tout = [tout0, tout1]
    fidx = [fidx0, fidx1]
    fout = [fout0, fout1]
    sem_wi = [sem_wi0, sem_wi1]
    sem_ti = [sem_ti0, sem_ti1]
    sem_sw = [sem_sw0, sem_sw1]
    sem_st = [sem_st0, sem_st1]

    pltpu.sync_copy(ttbl_hbm, ttbl_v)

    r0s = [row0 + j * _G for j in range(_NCHUNK)]

    d_ti = [None, None]
    d_wi = [None, None]
    d_g = [None, None]
    d_sw = [None, None]
    d_st = [None, None]

    def pack(widx_b, fidx_b):
        def body(r, c):
            for cs in _WINS:
                fidx_b[pl.ds(r * _COLS + cs, _L)] = widx_b[r, pl.ds(cs, _L)]
            return c
        lax.fori_loop(0, _G, body, 0)

    def unpack(fout_b, wout_b):
        def body(r, c):
            for cs in _WINS:
                wout_b[r, pl.ds(cs, _L)] = fout_b[pl.ds(r * _COLS + cs, _L)]
            return c
        lax.fori_loop(0, _G, body, 0)

    def tags(tidx_b, tout_b):
        def body(r, c):
            for cs in _WINS:
                idx = tidx_b[r, pl.ds(cs, _L)]
                tout_b[r, pl.ds(cs, _L)] = plsc.load_gather(ttbl_v, [idx])
            return c
        lax.fori_loop(0, _G, body, 0)

    for s in range(min(2, _NCHUNK)):
        d_wi[s] = pltpu.async_copy(sent_hbm.at[pl.ds(r0s[s], _G), :],
                                   widx[s], sem_wi[s])
        d_ti[s] = pltpu.async_copy(tagids_hbm.at[pl.ds(r0s[s], _G), :],
                                   tidx[s], sem_ti[s])
    d_wi[0].wait()
    d_wi[0] = None
    pack(widx[0], fidx[0])
    d_g[0] = pltpu.async_copy(wtbl_hbm.at[fidx[0]], fout[0], sem_g)

    for j in range(_NCHUNK):
        b = j & 1
        nb = b ^ 1

        if j + 2 < _NCHUNK:
            d_wi[b] = pltpu.async_copy(
                sent_hbm.at[pl.ds(r0s[j + 2], _G), :], widx[b], sem_wi[b])

        if j + 1 < _NCHUNK:
            d_wi[nb].wait()
            d_wi[nb] = None
            pack(widx[nb], fidx[nb])

        d_ti[b].wait()
        d_ti[b] = None
        if d_st[b] is not None:
            d_st[b].wait()
        tags(tidx[b], tout[b])
        d_st[b] = pltpu.async_copy(tout[b],
                                   out_tag_hbm.at[pl.ds(r0s[j], _G), :],
                                   sem_st[b])
        if j + 2 < _NCHUNK:
            d_ti[b] = pltpu.async_copy(
                tagids_hbm.at[pl.ds(r0s[j + 2], _G), :], tidx[b], sem_ti[b])

        d_g[b].wait()
        d_g[b] = None
        if d_sw[b] is not None:
            d_sw[b].wait()
        unpack(fout[b], wout[b])
        d_sw[b] = pltpu.async_copy(wout[b],
                                   out_tok_hbm.at[pl.ds(r0s[j], _G), :],
                                   sem_sw[b])

        if j + 1 < _NCHUNK:
            d_g[nb] = pltpu.async_copy(wtbl_hbm.at[fidx[nb]], fout[nb],
                                       sem_g)

    for d in (*d_sw, *d_st):
        if d is not None:
            d.wait()


def kernel(sentence_tensor, tag_string_tensor, word_table, tag_table):
    shape = sentence_tensor.shape

    mesh = plsc.VectorSubcoreMesh(core_axis_name="c", subcore_axis_name="s")
    k = pl.kernel(
        _body,
        mesh=mesh,
        compiler_params=pltpu.CompilerParams(needs_layout_passes=False),
        out_type=[
            jax.ShapeDtypeStruct(shape, jnp.int32),
            jax.ShapeDtypeStruct(shape, jnp.int32),
        ],
        scratch_types=[
            pltpu.VMEM((_G, _COLS), jnp.int32),
            pltpu.VMEM((_G, _COLS), jnp.int32),
            pltpu.VMEM((_G, _COLS), jnp.int32),
            pltpu.VMEM((_G, _COLS), jnp.int32),
            pltpu.VMEM((_G, _COLS), jnp.int32),
            pltpu.VMEM((_G, _COLS), jnp.int32),
            pltpu.VMEM((_G, _COLS), jnp.int32),
            pltpu.VMEM((_G, _COLS), jnp.int32),
            pltpu.VMEM((_FLAT,), jnp.int32),
            pltpu.VMEM((_FLAT,), jnp.int32),
            pltpu.VMEM((_FLAT,), jnp.int32),
            pltpu.VMEM((_FLAT,), jnp.int32),
            pltpu.VMEM((64,), jnp.int32),
        ] + [pltpu.SemaphoreType.DMA] * 9,
    )
    tok, tag = k(sentence_tensor, tag_string_tensor, word_table, tag_table)
    return (tok, tag)


# speedup vs baseline: 190.2036x; 1.4852x over previous
import jax
import jax.numpy as jnp
from jax import lax
from jax.experimental import pallas as pl
from jax.experimental.pallas import tpu as pltpu, tpu_sc as plsc

_info = plsc.get_sparse_core_info()
_NC, _NS, _L = _info.num_cores, _info.num_subcores, _info.num_lanes
_NW = _NC * _NS

_ROWS = 16384
_COLS = 200
_ROWS_W = _ROWS // _NW
_G = 32
_NCHUNK = _ROWS_W // _G
_FLAT = _G * _COLS

_WINS = [16 * w for w in range(12)] + [_COLS - _L]

_WTBL = 1001000
_STAGE = 62560
_SROUND = 3128
_NSROUND = _STAGE // _SROUND
_STAGE_TAIL = _WTBL - _NS * _STAGE


def _body(sent_hbm, tagids_hbm, wtbl_hbm, ttbl_hbm, out_tok_hbm, out_tag_hbm,
          widx0, widx1, wout_v, tidx_v, tout_v,
          fidx0, fidx1, fout0, fout1, ttbl_v, stbl,
          sem_wi0, sem_wi1, sem_ti, sem_g, sem_sw, sem_st,
          sem_s0, sem_s1, sem_s2, sem_s3):
    wid = lax.axis_index("s") * _NC + lax.axis_index("c")
    row0 = wid * _ROWS_W
    sid = lax.axis_index("s")

    widx = [widx0, widx1]
    fidx = [fidx0, fidx1]
    fout = [fout0, fout1]
    sem_wi = [sem_wi0, sem_wi1]

    pltpu.sync_copy(ttbl_hbm, ttbl_v)

    r0s = [row0 + j * _G for j in range(_NCHUNK)]

    d_ti = None
    d_wi = [None, None]
    d_g = [None, None]
    d_sw = None
    d_st = None

    def pack(widx_b, fidx_b):
        def body(r, c):
            for cs in _WINS:
                fidx_b[pl.ds(r * _COLS + cs, _L)] = widx_b[r, pl.ds(cs, _L)]
            return c
        lax.fori_loop(0, _G, body, 0)

    def unpack(fout_b):
        def body(r, c):
            for cs in _WINS:
                wout_v[r, pl.ds(cs, _L)] = fout_b[pl.ds(r * _COLS + cs, _L)]
            return c
        lax.fori_loop(0, _G, body, 0)

    def tags():
        def body(r, c):
            for cs in _WINS:
                idx = tidx_v[r, pl.ds(cs, _L)]
                tout_v[r, pl.ds(cs, _L)] = plsc.load_gather(ttbl_v, [idx])
            return c
        lax.fori_loop(0, _G, body, 0)

    d_wi[0] = pltpu.async_copy(sent_hbm.at[pl.ds(r0s[0], _G), :], widx[0],
                               sem_wi[0])
    d_wi[1] = pltpu.async_copy(sent_hbm.at[pl.ds(r0s[1], _G), :], widx[1],
                               sem_wi[1])
    d_ti = pltpu.async_copy(tagids_hbm.at[pl.ds(r0s[0], _G), :], tidx_v,
                            sem_ti)

    sbase = sid * _STAGE
    fbuf = [fout[0].at[pl.ds(0, _SROUND)], fout[1].at[pl.ds(0, _SROUND)]]
    sem_si = [sem_s0, sem_s1]
    sem_so = [sem_s2, sem_s3]
    din = [None, None]
    dout = [None, None]
    din[0] = pltpu.async_copy(wtbl_hbm.at[pl.ds(sbase, _SROUND)], fbuf[0],
                              sem_si[0])
    for r in range(_NSROUND):
        a = r & 1
        na = a ^ 1
        if r + 1 < _NSROUND:
            din[na] = pltpu.async_copy(
                wtbl_hbm.at[pl.ds(sbase + (r + 1) * _SROUND, _SROUND)],
                fbuf[na], sem_si[na])
        din[a].wait()
        if dout[a] is not None:
            dout[a].wait()
        dout[a] = pltpu.async_copy(
            fbuf[a], stbl.at[pl.ds(sbase + r * _SROUND, _SROUND)], sem_so[a])
    for d in dout:
        if d is not None:
            d.wait()

    @pl.when(sid == _NS - 1)
    def _():
        pltpu.sync_copy(wtbl_hbm.at[pl.ds(_NS * _STAGE, _STAGE_TAIL)],
                        fbuf[0].at[pl.ds(0, _STAGE_TAIL)])
        pltpu.sync_copy(fbuf[0].at[pl.ds(0, _STAGE_TAIL)],
                        stbl.at[pl.ds(_NS * _STAGE, _STAGE_TAIL)])

    d_wi[0].wait()
    d_wi[0] = None
    pack(widx[0], fidx[0])
    plsc.subcore_barrier()
    d_g[0] = pltpu.async_copy(stbl.at[fidx[0]], fout[0], sem_g)

    for j in range(_NCHUNK):
        b = j & 1
        nb = b ^ 1

        if j + 2 < _NCHUNK:
            d_wi[b] = pltpu.async_copy(
                sent_hbm.at[pl.ds(r0s[j + 2], _G), :], widx[b], sem_wi[b])

        if j + 1 < _NCHUNK:
            d_wi[nb].wait()
            d_wi[nb] = None
            pack(widx[nb], fidx[nb])

        d_ti.wait()
        if d_st is not None:
            d_st.wait()
        tags()
        d_st = pltpu.async_copy(tout_v, out_tag_hbm.at[pl.ds(r0s[j], _G), :],
                                sem_st)
        if j + 1 < _NCHUNK:
            d_ti = pltpu.async_copy(
                tagids_hbm.at[pl.ds(r0s[j + 1], _G), :], tidx_v, sem_ti)

        d_g[b].wait()
        d_g[b] = None
        if d_sw is not None:
            d_sw.wait()
        unpack(fout[b])
        d_sw = pltpu.async_copy(wout_v, out_tok_hbm.at[pl.ds(r0s[j], _G), :],
                                sem_sw)

        if j + 1 < _NCHUNK:
            d_g[nb] = pltpu.async_copy(stbl.at[fidx[nb]], fout[nb], sem_g)

    for d in (d_sw, d_st):
        if d is not None:
            d.wait()


def kernel(sentence_tensor, tag_string_tensor, word_table, tag_table):
    shape = sentence_tensor.shape

    mesh = plsc.VectorSubcoreMesh(core_axis_name="c", subcore_axis_name="s")
    k = pl.kernel(
        _body,
        mesh=mesh,
        compiler_params=pltpu.CompilerParams(needs_layout_passes=False),
        out_type=[
            jax.ShapeDtypeStruct(shape, jnp.int32),
            jax.ShapeDtypeStruct(shape, jnp.int32),
        ],
        scratch_types=[
            pltpu.VMEM((_G, _COLS), jnp.int32),
            pltpu.VMEM((_G, _COLS), jnp.int32),
            pltpu.VMEM((_G, _COLS), jnp.int32),
            pltpu.VMEM((_G, _COLS), jnp.int32),
            pltpu.VMEM((_G, _COLS), jnp.int32),
            pltpu.VMEM((_FLAT,), jnp.int32),
            pltpu.VMEM((_FLAT,), jnp.int32),
            pltpu.VMEM((_FLAT,), jnp.int32),
            pltpu.VMEM((_FLAT,), jnp.int32),
            pltpu.VMEM((64,), jnp.int32),
            pltpu.VMEM_SHARED((_WTBL,), jnp.int32),
        ] + [pltpu.SemaphoreType.DMA] * 10,
    )
    tok, tag = k(sentence_tensor, tag_string_tensor, word_table, tag_table)
    return (tok, tag)


# speedup vs baseline: 302.5364x; 1.5906x over previous
import jax
import jax.numpy as jnp
from jax import lax
from jax.experimental import pallas as pl
from jax.experimental.pallas import tpu as pltpu, tpu_sc as plsc

_info = plsc.get_sparse_core_info()
_NC, _NS, _L = _info.num_cores, _info.num_subcores, _info.num_lanes
_NW = _NC * _NS

_R = 200
_C = 16384
_COLS_W = _C // _NW
_GR = 40
_GC = 128
_NRC = _R // _GR
_NCC = _COLS_W // _GC
_NCHUNK = _NRC * _NCC
_FLAT = _GR * _GC

_WTBL = 1001000
_STAGE = 62560
_SROUND = 3128
_NSROUND = _STAGE // _SROUND
_STAGE_TAIL = _WTBL - _NS * _STAGE


def _body(sent_hbm, tagids_hbm, wtbl_hbm, ttbl_hbm, out_tok_hbm, out_tag_hbm,
          widx0, widx1, wout_v, tidx_v, tout_v,
          fidx0, fidx1, fout0, fout1, ttbl_v, stbl,
          sem_wi0, sem_wi1, sem_ti, sem_g, sem_sw, sem_st,
          sem_s0, sem_s1, sem_s2, sem_s3):
    wid = lax.axis_index("s") * _NC + lax.axis_index("c")
    col0 = wid * _COLS_W
    sid = lax.axis_index("s")

    widx = [widx0, widx1]
    fidx = [fidx0, fidx1]
    fout = [fout0, fout1]
    sem_wi = [sem_wi0, sem_wi1]

    pltpu.sync_copy(ttbl_hbm, ttbl_v)

    orgs = [(rc * _GR, col0 + cc * _GC)
            for cc in range(_NCC) for rc in range(_NRC)]

    d_ti = None
    d_wi = [None, None]
    d_g = [None, None]
    d_sw = None
    d_st = None

    def pack(widx_b, fidx_b):
        def body(r, c):
            for w in range(_GC // _L):
                cs = w * _L
                fidx_b[pl.ds(r * _GC + cs, _L)] = widx_b[r, pl.ds(cs, _L)]
            return c
        lax.fori_loop(0, _GR, body, 0)

    def unpack(fout_b):
        def body(r, c):
            for w in range(_GC // _L):
                cs = w * _L
                wout_v[r, pl.ds(cs, _L)] = fout_b[pl.ds(r * _GC + cs, _L)]
            return c
        lax.fori_loop(0, _GR, body, 0)

    def tags():
        def body(r, c):
            for w in range(_GC // _L):
                cs = w * _L
                idx = tidx_v[r, pl.ds(cs, _L)]
                tout_v[r, pl.ds(cs, _L)] = plsc.load_gather(ttbl_v, [idx])
            return c
        lax.fori_loop(0, _GR, body, 0)

    def win(hbm, j):
        r0, c0 = orgs[j]
        return hbm.at[pl.ds(r0, _GR), pl.ds(c0, _GC)]

    d_wi[0] = pltpu.async_copy(win(sent_hbm, 0), widx[0], sem_wi[0])
    d_wi[1] = pltpu.async_copy(win(sent_hbm, 1), widx[1], sem_wi[1])
    d_ti = pltpu.async_copy(win(tagids_hbm, 0), tidx_v, sem_ti)

    sbase = sid * _STAGE
    fbuf = [fout[0].at[pl.ds(0, _SROUND)], fout[1].at[pl.ds(0, _SROUND)]]
    sem_si = [sem_s0, sem_s1]
    sem_so = [sem_s2, sem_s3]
    din = [None, None]
    dout = [None, None]
    din[0] = pltpu.async_copy(wtbl_hbm.at[pl.ds(sbase, _SROUND)], fbuf[0],
                              sem_si[0])
    for r in range(_NSROUND):
        a = r & 1
        na = a ^ 1
        if r + 1 < _NSROUND:
            din[na] = pltpu.async_copy(
                wtbl_hbm.at[pl.ds(sbase + (r + 1) * _SROUND, _SROUND)],
                fbuf[na], sem_si[na])
        din[a].wait()
        if dout[a] is not None:
            dout[a].wait()
        dout[a] = pltpu.async_copy(
            fbuf[a], stbl.at[pl.ds(sbase + r * _SROUND, _SROUND)], sem_so[a])
    for d in dout:
        if d is not None:
            d.wait()

    @pl.when(sid == _NS - 1)
    def _():
        pltpu.sync_copy(wtbl_hbm.at[pl.ds(_NS * _STAGE, _STAGE_TAIL)],
                        fbuf[0].at[pl.ds(0, _STAGE_TAIL)])
        pltpu.sync_copy(fbuf[0].at[pl.ds(0, _STAGE_TAIL)],
                        stbl.at[pl.ds(_NS * _STAGE, _STAGE_TAIL)])

    def gtbl(j):
        return wtbl_hbm if j % 4 == 3 else stbl

    d_wi[0].wait()
    d_wi[0] = None
    pack(widx[0], fidx[0])
    plsc.subcore_barrier()
    d_g[0] = pltpu.async_copy(gtbl(0).at[fidx[0]], fout[0], sem_g)

    for j in range(_NCHUNK):
        b = j & 1
        nb = b ^ 1

        if j + 2 < _NCHUNK:
            d_wi[b] = pltpu.async_copy(win(sent_hbm, j + 2), widx[b],
                                       sem_wi[b])

        if j + 1 < _NCHUNK:
            d_wi[nb].wait()
            d_wi[nb] = None
            pack(widx[nb], fidx[nb])

        d_ti.wait()
        if d_st is not None:
            d_st.wait()
        tags()
        d_st = pltpu.async_copy(tout_v, win(out_tag_hbm, j), sem_st)
        if j + 1 < _NCHUNK:
            d_ti = pltpu.async_copy(win(tagids_hbm, j + 1), tidx_v, sem_ti)

        d_g[b].wait()
        d_g[b] = None
        if d_sw is not None:
            d_sw.wait()
        unpack(fout[b])
        d_sw = pltpu.async_copy(wout_v, win(out_tok_hbm, j), sem_sw)

        if j + 1 < _NCHUNK:
            d_g[nb] = pltpu.async_copy(gtbl(j + 1).at[fidx[nb]], fout[nb],
                                       sem_g)

    for d in (d_sw, d_st):
        if d is not None:
            d.wait()


def kernel(sentence_tensor, tag_string_tensor, word_table, tag_table):
    mesh = plsc.VectorSubcoreMesh(core_axis_name="c", subcore_axis_name="s")
    k = pl.kernel(
        _body,
        mesh=mesh,
        compiler_params=pltpu.CompilerParams(needs_layout_passes=False),
        out_type=[
            jax.ShapeDtypeStruct((_R, _C), jnp.int32),
            jax.ShapeDtypeStruct((_R, _C), jnp.int32),
        ],
        scratch_types=[
            pltpu.VMEM((_GR, _GC), jnp.int32),
            pltpu.VMEM((_GR, _GC), jnp.int32),
            pltpu.VMEM((_GR, _GC), jnp.int32),
            pltpu.VMEM((_GR, _GC), jnp.int32),
            pltpu.VMEM((_GR, _GC), jnp.int32),
            pltpu.VMEM((_FLAT,), jnp.int32),
            pltpu.VMEM((_FLAT,), jnp.int32),
            pltpu.VMEM((_FLAT,), jnp.int32),
            pltpu.VMEM((_FLAT,), jnp.int32),
            pltpu.VMEM((64,), jnp.int32),
            pltpu.VMEM_SHARED((_WTBL,), jnp.int32),
        ] + [pltpu.SemaphoreType.DMA] * 10,
    )
    tok_t, tag_t = k(sentence_tensor.T, tag_string_tensor.T,
                     word_table, tag_table)
    return (tok_t.T, tag_t.T)


# speedup vs baseline: 373.0229x; 1.2330x over previous
import jax
import jax.numpy as jnp
from jax import lax
from jax.experimental import pallas as pl
from jax.experimental.pallas import tpu as pltpu, tpu_sc as plsc

_info = plsc.get_sparse_core_info()
_NC, _NS, _L = _info.num_cores, _info.num_subcores, _info.num_lanes
_NW = _NC * _NS

_R = 200
_C = 16384
_COLS_W = _C // _NW
_GR = 40
_GC = 128
_NRC = _R // _GR
_NCC = _COLS_W // _GC
_NCHUNK = _NRC * _NCC
_FLAT = _GR * _GC

_WTBL = 1001000
_STAGE = 62560
_SROUND = 3128
_NSROUND = _STAGE // _SROUND
_STAGE_TAIL = _WTBL - _NS * _STAGE


def _body(sent_hbm, tagids_hbm, wtbl_hbm, ttbl_hbm, out_tok_hbm, out_tag_hbm,
          widx0, widx1, wout_v, tidx_v, tout_v,
          fidx0, fidx1, fout0, fout1, ttbl_v, stbl,
          sem_wi0, sem_wi1, sem_ti, sem_g, sem_sw, sem_st,
          sem_s0, sem_s1, sem_s2, sem_s3):
    wid = lax.axis_index("s") * _NC + lax.axis_index("c")
    col0 = wid * _COLS_W
    sid = lax.axis_index("s")

    widx = [widx0, widx1]
    fidx = [fidx0, fidx1]
    fout = [fout0, fout1]
    sem_wi = [sem_wi0, sem_wi1]

    pltpu.sync_copy(ttbl_hbm, ttbl_v)

    orgs = [(rc * _GR, col0 + cc * _GC)
            for cc in range(_NCC) for rc in range(_NRC)]

    d_ti = None
    d_wi = [None, None]
    d_g = [None, None]
    d_sw = None
    d_st = None

    def pack(widx_b, fidx_b):
        def body(r, c):
            for w in range(_GC // _L):
                cs = w * _L
                fidx_b[pl.ds(r * _GC + cs, _L)] = widx_b[r, pl.ds(cs, _L)]
            return c
        lax.fori_loop(0, _GR, body, 0)

    def unpack(fout_b):
        def body(r, c):
            for w in range(_GC // _L):
                cs = w * _L
                wout_v[r, pl.ds(cs, _L)] = fout_b[pl.ds(r * _GC + cs, _L)]
            return c
        lax.fori_loop(0, _GR, body, 0)

    def tags():
        def body(r, c):
            for w in range(_GC // _L):
                cs = w * _L
                idx = tidx_v[r, pl.ds(cs, _L)]
                tout_v[r, pl.ds(cs, _L)] = plsc.load_gather(ttbl_v, [idx])
            return c
        lax.fori_loop(0, _GR, body, 0)

    def win(hbm, j):
        r0, c0 = orgs[j]
        return hbm.at[pl.ds(r0, _GR), pl.ds(c0, _GC)]

    d_wi[0] = pltpu.async_copy(win(sent_hbm, 0), widx[0], sem_wi[0])
    d_wi[1] = pltpu.async_copy(win(sent_hbm, 1), widx[1], sem_wi[1])
    d_ti = pltpu.async_copy(win(tagids_hbm, 0), tidx_v, sem_ti)

    sbase = sid * _STAGE
    fbuf = [fout[0].at[pl.ds(0, _SROUND)], fout[1].at[pl.ds(0, _SROUND)]]
    sem_si = [sem_s0, sem_s1]
    sem_so = [sem_s2, sem_s3]
    din = [None, None]
    dout = [None, None]
    din[0] = pltpu.async_copy(wtbl_hbm.at[pl.ds(sbase, _SROUND)], fbuf[0],
                              sem_si[0])
    for r in range(_NSROUND):
        a = r & 1
        na = a ^ 1
        if r + 1 < _NSROUND:
            din[na] = pltpu.async_copy(
                wtbl_hbm.at[pl.ds(sbase + (r + 1) * _SROUND, _SROUND)],
                fbuf[na], sem_si[na])
        din[a].wait()
        if dout[a] is not None:
            dout[a].wait()
        dout[a] = pltpu.async_copy(
            fbuf[a], stbl.at[pl.ds(sbase + r * _SROUND, _SROUND)], sem_so[a])
    for d in dout:
        if d is not None:
            d.wait()

    @pl.when(sid == _NS - 1)
    def _():
        pltpu.sync_copy(wtbl_hbm.at[pl.ds(_NS * _STAGE, _STAGE_TAIL)],
                        fbuf[0].at[pl.ds(0, _STAGE_TAIL)])
        pltpu.sync_copy(fbuf[0].at[pl.ds(0, _STAGE_TAIL)],
                        stbl.at[pl.ds(_NS * _STAGE, _STAGE_TAIL)])

    d_wi[0].wait()
    d_wi[0] = None
    pack(widx[0], fidx[0])
    plsc.subcore_barrier()
    d_g[0] = pltpu.async_copy(stbl.at[fidx[0]], fout[0], sem_g)

    for j in range(_NCHUNK):
        b = j & 1
        nb = b ^ 1

        if j + 2 < _NCHUNK:
            d_wi[b] = pltpu.async_copy(win(sent_hbm, j + 2), widx[b],
                                       sem_wi[b])

        if j + 1 < _NCHUNK:
            d_wi[nb].wait()
            d_wi[nb] = None
            pack(widx[nb], fidx[nb])

        d_ti.wait()
        if d_st is not None:
            d_st.wait()
        tags()
        d_st = pltpu.async_copy(tout_v, win(out_tag_hbm, j), sem_st)
        if j + 1 < _NCHUNK:
            d_ti = pltpu.async_copy(win(tagids_hbm, j + 1), tidx_v, sem_ti)

        d_g[b].wait()
        d_g[b] = None
        if d_sw is not None:
            d_sw.wait()
        unpack(fout[b])
        d_sw = pltpu.async_copy(wout_v, win(out_tok_hbm, j), sem_sw)

        if j + 1 < _NCHUNK:
            d_g[nb] = pltpu.async_copy(stbl.at[fidx[nb]], fout[nb], sem_g)

    for d in (d_sw, d_st):
        if d is not None:
            d.wait()


def kernel(sentence_tensor, tag_string_tensor, word_table, tag_table):
    mesh = plsc.VectorSubcoreMesh(core_axis_name="c", subcore_axis_name="s")
    k = pl.kernel(
        _body,
        mesh=mesh,
        compiler_params=pltpu.CompilerParams(needs_layout_passes=False),
        out_type=[
            jax.ShapeDtypeStruct((_R, _C), jnp.int32),
            jax.ShapeDtypeStruct((_R, _C), jnp.int32),
        ],
        scratch_types=[
            pltpu.VMEM((_GR, _GC), jnp.int32),
            pltpu.VMEM((_GR, _GC), jnp.int32),
            pltpu.VMEM((_GR, _GC), jnp.int32),
            pltpu.VMEM((_GR, _GC), jnp.int32),
            pltpu.VMEM((_GR, _GC), jnp.int32),
            pltpu.VMEM((_FLAT,), jnp.int32),
            pltpu.VMEM((_FLAT,), jnp.int32),
            pltpu.VMEM((_FLAT,), jnp.int32),
            pltpu.VMEM((_FLAT,), jnp.int32),
            pltpu.VMEM((64,), jnp.int32),
            pltpu.VMEM_SHARED((_WTBL,), jnp.int32),
        ] + [pltpu.SemaphoreType.DMA] * 10,
    )
    tok_t, tag_t = k(sentence_tensor.T, tag_string_tensor.T,
                     word_table, tag_table)
    return (tok_t.T, tag_t.T)
